# Initial kernel scaffold; baseline (speedup 1.0000x reference)
import jax
import jax.numpy as jnp
from jax.experimental import pallas as pl


def kernel(x, edge_index, edge_attr, batch, W0, b0, We1, be1, We2, be2, Wroot, bconv, Wih, bih, Whh, bhh, W11, b11, W12, b12, W13, b13, W21, b21, W22, b22, W23, b23):
    raise NotImplementedError("write your pallas kernel here")


# speedup vs baseline: 3.6644x; 3.6644x over previous
import functools

import jax
import jax.numpy as jnp
from jax import lax
from jax.experimental import pallas as pl
from jax.experimental.pallas import tpu as pltpu
from jax.experimental.pallas import tpu_sc as plsc


def _lin0_body(x_ref, w_ref, b_ref, o_ref):
    o_ref[...] = jnp.maximum(
        jnp.dot(x_ref[...], w_ref[...], preferred_element_type=jnp.float32)
        + b_ref[...], 0.0)


def _lin0(x, w, b):
    n, f = x.shape
    d = w.shape[1]
    bn = 1000
    return pl.pallas_call(
        _lin0_body,
        grid=(n // bn,),
        in_specs=[pl.BlockSpec((bn, f), lambda i: (i, 0)),
                  pl.BlockSpec((f, d), lambda i: (0, 0)),
                  pl.BlockSpec((1, d), lambda i: (0, 0))],
        out_specs=pl.BlockSpec((bn, d), lambda i: (i, 0)),
        out_shape=jax.ShapeDtypeStruct((n, d), jnp.float32),
    )(x, w, b.reshape(1, d))


def _sc_gather(table, idx):
    e = idx.shape[0]
    d = table.shape[1]
    nw = 32
    bpw = e // nw
    mesh = plsc.VectorSubcoreMesh(core_axis_name="c", subcore_axis_name="s")

    @functools.partial(
        pl.kernel, mesh=mesh,
        out_type=jax.ShapeDtypeStruct((e, d), jnp.float32),
        compiler_params=pltpu.CompilerParams(use_tc_tiling_on_sc=False),
        scratch_types=[pltpu.VMEM((bpw,), jnp.int32),
                       pltpu.VMEM((bpw, d), jnp.float32),
                       pltpu.SemaphoreType.DMA],
    )
    def k(table_hbm, idx_hbm, out_hbm, idx_v, rows_v, sem):
        wid = lax.axis_index("s") * 2 + lax.axis_index("c")
        base = wid * bpw
        pltpu.sync_copy(idx_hbm.at[pl.ds(base, bpw)], idx_v)
        pltpu.async_copy(table_hbm.at[idx_v], rows_v, sem).wait()
        pltpu.sync_copy(rows_v, out_hbm.at[pl.ds(base, bpw)])

    return k(table, idx)


def _sc_scatter_add(msg, dst, n):
    e, d = msg.shape
    nw, ns = 32, 16
    bpw = e // nw
    nps = n // ns
    mesh = plsc.VectorSubcoreMesh(core_axis_name="c", subcore_axis_name="s")

    @functools.partial(
        pl.kernel, mesh=mesh,
        out_type=jax.ShapeDtypeStruct((2, n, d), jnp.float32),
        compiler_params=pltpu.CompilerParams(use_tc_tiling_on_sc=False),
        scratch_types=[pltpu.VMEM((bpw,), jnp.int32),
                       pltpu.VMEM((bpw, d), jnp.float32),
                       pltpu.VMEM((nps, d), jnp.float32),
                       pltpu.VMEM_SHARED((n, d), jnp.float32),
                       pltpu.SemaphoreType.DMA],
    )
    def k(msg_hbm, dst_hbm, zeros_hbm, out_hbm, idx_v, rows_v, z_v, acc_sh,
          sem):
        cid = lax.axis_index("c")
        sid = lax.axis_index("s")
        wid = sid * 2 + cid
        pltpu.sync_copy(zeros_hbm.at[pl.ds(sid * nps, nps)], z_v)
        pltpu.sync_copy(z_v, acc_sh.at[pl.ds(sid * nps, nps)])
        plsc.subcore_barrier()
        base = wid * bpw
        pltpu.sync_copy(dst_hbm.at[pl.ds(base, bpw)], idx_v)
        pltpu.sync_copy(msg_hbm.at[pl.ds(base, bpw)], rows_v)
        pltpu.sync_copy(rows_v, acc_sh.at[idx_v], add=True)
        plsc.subcore_barrier()
        pltpu.sync_copy(acc_sh.at[pl.ds(sid * nps, nps)],
                        out_hbm.at[cid, pl.ds(sid * nps, nps)])

    return k(msg, dst, jnp.zeros((n, d), jnp.float32))


def _edge_body(ea_ref, xs_ref, we1_ref, be1_ref, w2r_ref, be2m_ref, o_ref):
    ea = ea_ref[...]
    xs = xs_ref[...]
    eh = jnp.maximum(
        jnp.dot(ea, we1_ref[...], preferred_element_type=jnp.float32)
        + be1_ref[...], 0.0)
    col = lax.broadcasted_iota(jnp.int32, (16, 256), 1)
    row = lax.broadcasted_iota(jnp.int32, (16, 256), 0)
    kmat = (row == col // 16).astype(jnp.float32)
    lmat = (row == col % 16).astype(jnp.float32)
    z = (jnp.dot(eh, kmat, preferred_element_type=jnp.float32)
         * jnp.dot(xs, lmat, preferred_element_type=jnp.float32))
    o_ref[...] = (jnp.dot(z, w2r_ref[...], preferred_element_type=jnp.float32)
                  + jnp.dot(xs, be2m_ref[...],
                            preferred_element_type=jnp.float32))


def _edge(edge_attr, x0s, we1, be1, w2r, be2m):
    e, fe = edge_attr.shape
    d = we1.shape[1]
    be = 2000
    return pl.pallas_call(
        _edge_body,
        grid=(e // be,),
        in_specs=[pl.BlockSpec((be, fe), lambda i: (i, 0)),
                  pl.BlockSpec((be, d), lambda i: (i, 0)),
                  pl.BlockSpec((fe, d), lambda i: (0, 0)),
                  pl.BlockSpec((1, d), lambda i: (0, 0)),
                  pl.BlockSpec((d * d, d), lambda i: (0, 0)),
                  pl.BlockSpec((d, d), lambda i: (0, 0))],
        out_specs=pl.BlockSpec((be, d), lambda i: (i, 0)),
        out_shape=jax.ShapeDtypeStruct((e, d), jnp.float32),
    )(edge_attr, x0s, we1, be1.reshape(1, d), w2r, be2m)


def _node_body(x0_ref, agg_ref, b3_ref, wroot_ref, bconv_ref, wih_ref,
               bih_ref, whh_ref, bhh_ref, w11_ref, b11_ref, w12_ref, b12_ref,
               w13_ref, b13_ref, w21_ref, b21_ref, w22_ref, b22_ref, w23_ref,
               b23_ref, o_ref, acc_ref):
    i = pl.program_id(0)
    x0 = x0_ref[...]
    agg = agg_ref[0] + agg_ref[1]
    xc = jnp.maximum(
        jnp.dot(x0, wroot_ref[...], preferred_element_type=jnp.float32)
        + agg + bconv_ref[...], 0.0)
    gi = jnp.dot(xc, wih_ref[...], preferred_element_type=jnp.float32) \
        + bih_ref[...]
    gh = jnp.dot(x0, whh_ref[...], preferred_element_type=jnp.float32) \
        + bhh_ref[...]
    r = jax.nn.sigmoid(gi[:, 0:16] + gh[:, 0:16])
    zg = jax.nn.sigmoid(gi[:, 16:32] + gh[:, 16:32])
    ng = jnp.tanh(gi[:, 32:48] + r * gh[:, 32:48])
    hn = (1.0 - zg) * ng + zg * x0
    bvals = b3_ref[0]
    g_iota = lax.broadcasted_iota(jnp.int32, (256, bvals.shape[1]), 0)
    onehot = (g_iota == bvals).astype(jnp.float32)
    part = jnp.dot(onehot, hn, preferred_element_type=jnp.float32)

    @pl.when(i == 0)
    def _():
        acc_ref[...] = part

    @pl.when(i > 0)
    def _():
        acc_ref[...] += part

    @pl.when(i == pl.num_programs(0) - 1)
    def _():
        p = acc_ref[...]
        x1 = jnp.maximum(
            jnp.dot(p, w11_ref[...], preferred_element_type=jnp.float32)
            + b11_ref[...], 0.0)
        x1 = jnp.maximum(
            jnp.dot(x1, w12_ref[...], preferred_element_type=jnp.float32)
            + b12_ref[...], 0.0)
        o1 = jnp.dot(x1, w13_ref[...], preferred_element_type=jnp.float32) \
            + b13_ref[...]
        x2 = jnp.maximum(
            jnp.dot(p, w21_ref[...], preferred_element_type=jnp.float32)
            + b21_ref[...], 0.0)
        x2 = jnp.maximum(
            jnp.dot(x2, w22_ref[...], preferred_element_type=jnp.float32)
            + b22_ref[...], 0.0)
        o2 = jnp.dot(x2, w23_ref[...], preferred_element_type=jnp.float32) \
            + b23_ref[...]
        o_ref[...] = jnp.concatenate([o1, o2], axis=1)


def _node(x0, agg2, batch, wroot, bconv, wih, bih, whh, bhh, w11, b11, w12,
          b12, w13, b13, w21, b21, w22, b22, w23, b23):
    n, d = x0.shape
    g = 256
    bn = 1000
    nb = n // bn
    batch3 = batch.reshape(nb, 1, bn)
    full = lambda shape: pl.BlockSpec(shape, lambda i: tuple(0 for _ in shape))
    return pl.pallas_call(
        _node_body,
        grid=(nb,),
        in_specs=[pl.BlockSpec((bn, d), lambda i: (i, 0)),
                  pl.BlockSpec((2, bn, d), lambda i: (0, i, 0)),
                  pl.BlockSpec((1, 1, bn), lambda i: (i, 0, 0)),
                  full((d, d)), full((1, d)),
                  full((d, 3 * d)), full((1, 3 * d)),
                  full((d, 3 * d)), full((1, 3 * d)),
                  full((d, d)), full((1, d)),
                  full((d, d)), full((1, d)),
                  full((d, 1)), full((1, 1)),
                  full((d, d)), full((1, d)),
                  full((d, d)), full((1, d)),
                  full((d, 1)), full((1, 1))],
        out_specs=pl.BlockSpec((g, 2), lambda i: (0, 0)),
        out_shape=jax.ShapeDtypeStruct((g, 2), jnp.float32),
        scratch_shapes=[pltpu.VMEM((g, d), jnp.float32)],
    )(x0, agg2, batch3, wroot, bconv.reshape(1, d), wih, bih.reshape(1, 3 * d),
      whh, bhh.reshape(1, 3 * d), w11, b11.reshape(1, d), w12,
      b12.reshape(1, d), w13, b13.reshape(1, 1), w21, b21.reshape(1, d), w22,
      b22.reshape(1, d), w23, b23.reshape(1, 1))


def kernel(x, edge_index, edge_attr, batch, W0, b0, We1, be1, We2, be2, Wroot,
           bconv, Wih, bih, Whh, bhh, W11, b11, W12, b12, W13, b13, W21, b21,
           W22, b22, W23, b23):
    n = x.shape[0]
    d = W0.shape[1]
    x0 = _lin0(x, W0, b0)
    src = edge_index[0]
    dst = edge_index[1]
    x0s = _sc_gather(x0, src)
    msg = _edge(edge_attr, x0s, We1, be1, We2.reshape(d * d, d),
                be2.reshape(d, d))
    agg2 = _sc_scatter_add(msg, dst, n)
    return _node(x0, agg2, batch, Wroot, bconv, Wih, bih, Whh, bhh, W11, b11,
                 W12, b12, W13, b13, W21, b21, W22, b22, W23, b23)


# speedup vs baseline: 4.3004x; 1.1736x over previous
import functools

import jax
import jax.numpy as jnp
from jax import lax
from jax.experimental import pallas as pl
from jax.experimental.pallas import tpu as pltpu
from jax.experimental.pallas import tpu_sc as plsc


def _lin0_body(x_ref, w_ref, b_ref, o_ref):
    r = jnp.maximum(
        jnp.dot(x_ref[...], w_ref[...], preferred_element_type=jnp.float32)
        + b_ref[...], 0.0)
    bn, d = r.shape
    o_ref[...] = r.reshape(bn // 8, 8, d)


def _lin0(x, w, b):
    n, f = x.shape
    d = w.shape[1]
    return pl.pallas_call(
        _lin0_body,
        out_shape=jax.ShapeDtypeStruct((n // 8, 8, d), jnp.float32),
    )(x, w, b.reshape(1, d))


def _sc_gather(table, idx):
    e = idx.shape[0]
    d = table.shape[1]
    nw = 32
    bpw = e // nw
    mesh = plsc.VectorSubcoreMesh(core_axis_name="c", subcore_axis_name="s")

    @functools.partial(
        pl.kernel, mesh=mesh,
        out_type=jax.ShapeDtypeStruct((e, d), jnp.float32),
        compiler_params=pltpu.CompilerParams(use_tc_tiling_on_sc=False),
        scratch_types=[pltpu.VMEM((bpw,), jnp.int32),
                       pltpu.VMEM((bpw, d), jnp.float32),
                       pltpu.SemaphoreType.DMA],
    )
    def k(table_hbm, idx_hbm, out_hbm, idx_v, rows_v, sem):
        wid = lax.axis_index("s") * 2 + lax.axis_index("c")
        base = wid * bpw
        pltpu.sync_copy(idx_hbm.at[pl.ds(base, bpw)], idx_v)
        pltpu.async_copy(table_hbm.at[idx_v], rows_v, sem).wait()
        pltpu.sync_copy(rows_v, out_hbm.at[pl.ds(base, bpw)])

    return k(table, idx)


def _sc_scatter_add(msg, dst, n):
    e, d = msg.shape
    nw, ns = 32, 16
    bpw = e // nw
    nps = n // ns
    mesh = plsc.VectorSubcoreMesh(core_axis_name="c", subcore_axis_name="s")

    @functools.partial(
        pl.kernel, mesh=mesh,
        out_type=jax.ShapeDtypeStruct((2, n, d), jnp.float32),
        compiler_params=pltpu.CompilerParams(use_tc_tiling_on_sc=False),
        scratch_types=[pltpu.VMEM((bpw,), jnp.int32),
                       pltpu.VMEM((bpw, d), jnp.float32),
                       pltpu.VMEM((nps, d), jnp.float32),
                       pltpu.VMEM_SHARED((n, d), jnp.float32),
                       pltpu.SemaphoreType.DMA],
    )
    def k(msg_hbm, dst_hbm, zeros_hbm, out_hbm, idx_v, rows_v, z_v, acc_sh,
          sem):
        cid = lax.axis_index("c")
        sid = lax.axis_index("s")
        wid = sid * 2 + cid
        pltpu.sync_copy(zeros_hbm.at[pl.ds(sid * nps, nps)], z_v)
        pltpu.sync_copy(z_v, acc_sh.at[pl.ds(sid * nps, nps)])
        plsc.subcore_barrier()
        base = wid * bpw
        pltpu.sync_copy(dst_hbm.at[pl.ds(base, bpw)], idx_v)
        pltpu.sync_copy(msg_hbm.at[pl.ds(base, bpw)], rows_v)
        pltpu.sync_copy(rows_v, acc_sh.at[idx_v], add=True)
        plsc.subcore_barrier()
        pltpu.sync_copy(acc_sh.at[pl.ds(sid * nps, nps)],
                        out_hbm.at[cid, pl.ds(sid * nps, nps)])

    return k(msg, dst, jnp.zeros((n, d), jnp.float32))


def _edge_body(ea8_ref, xs8_ref, we1_ref, be1_ref, w2r_ref, be2m_ref, o_ref):
    r8 = ea8_ref.shape[0]
    be = r8 * 8
    ea = ea8_ref[...].reshape(be, 16)
    xs = xs8_ref[...].reshape(be, 16)
    eh = jnp.maximum(
        jnp.dot(ea, we1_ref[...], preferred_element_type=jnp.float32)
        + be1_ref[...], 0.0)
    col = lax.broadcasted_iota(jnp.int32, (16, 256), 1)
    row = lax.broadcasted_iota(jnp.int32, (16, 256), 0)
    kmat = (row == col // 16).astype(jnp.float32)
    lmat = (row == col % 16).astype(jnp.float32)
    z = (jnp.dot(eh, kmat, preferred_element_type=jnp.float32)
         * jnp.dot(xs, lmat, preferred_element_type=jnp.float32))
    msg = (jnp.dot(z, w2r_ref[...], preferred_element_type=jnp.float32)
           + jnp.dot(xs, be2m_ref[...], preferred_element_type=jnp.float32))
    o_ref[...] = msg.reshape(r8, 8, 16)


def _edge(ea3, xs3, we1, be1, w2r, be2m):
    e8 = ea3.shape[0]
    d = we1.shape[1]
    r8 = 400
    return pl.pallas_call(
        _edge_body,
        grid=(e8 // r8,),
        in_specs=[pl.BlockSpec((r8, 8, 16), lambda i: (i, 0, 0)),
                  pl.BlockSpec((r8, 8, 16), lambda i: (i, 0, 0)),
                  pl.BlockSpec((16, d), lambda i: (0, 0)),
                  pl.BlockSpec((1, d), lambda i: (0, 0)),
                  pl.BlockSpec((d * d, d), lambda i: (0, 0)),
                  pl.BlockSpec((d, d), lambda i: (0, 0))],
        out_specs=pl.BlockSpec((r8, 8, 16), lambda i: (i, 0, 0)),
        out_shape=jax.ShapeDtypeStruct((e8, 8, 16), jnp.float32),
    )(ea3, xs3, we1, be1.reshape(1, d), w2r, be2m)


def _node_body(x0_ref, agg_ref, b2_ref, wroot_ref, bconv_ref, wih_ref,
               bih_ref, whh_ref, bhh_ref, w11_ref, b11_ref, w12_ref, b12_ref,
               w13_ref, b13_ref, w21_ref, b21_ref, w22_ref, b22_ref, w23_ref,
               b23_ref, o_ref):
    r8 = x0_ref.shape[0]
    bn = r8 * 8
    x0 = x0_ref[...].reshape(bn, 16)
    agg = agg_ref[0].reshape(bn, 16) + agg_ref[1].reshape(bn, 16)
    xc = jnp.maximum(
        jnp.dot(x0, wroot_ref[...], preferred_element_type=jnp.float32)
        + agg + bconv_ref[...], 0.0)
    gi = jnp.dot(xc, wih_ref[...], preferred_element_type=jnp.float32) \
        + bih_ref[...]
    gh = jnp.dot(x0, whh_ref[...], preferred_element_type=jnp.float32) \
        + bhh_ref[...]
    r = jax.nn.sigmoid(gi[:, 0:16] + gh[:, 0:16])
    zg = jax.nn.sigmoid(gi[:, 16:32] + gh[:, 16:32])
    ng = jnp.tanh(gi[:, 32:48] + r * gh[:, 32:48])
    hn = (1.0 - zg) * ng + zg * x0
    bvals = b2_ref[...]
    g_iota = lax.broadcasted_iota(jnp.int32, (256, bn), 0)
    onehot = (g_iota == bvals).astype(jnp.float32)
    p = jnp.dot(onehot, hn, preferred_element_type=jnp.float32)
    x1 = jnp.maximum(
        jnp.dot(p, w11_ref[...], preferred_element_type=jnp.float32)
        + b11_ref[...], 0.0)
    x1 = jnp.maximum(
        jnp.dot(x1, w12_ref[...], preferred_element_type=jnp.float32)
        + b12_ref[...], 0.0)
    o1 = jnp.dot(x1, w13_ref[...], preferred_element_type=jnp.float32) \
        + b13_ref[...]
    x2 = jnp.maximum(
        jnp.dot(p, w21_ref[...], preferred_element_type=jnp.float32)
        + b21_ref[...], 0.0)
    x2 = jnp.maximum(
        jnp.dot(x2, w22_ref[...], preferred_element_type=jnp.float32)
        + b22_ref[...], 0.0)
    o2 = jnp.dot(x2, w23_ref[...], preferred_element_type=jnp.float32) \
        + b23_ref[...]
    o_ref[...] = jnp.concatenate([o1, o2], axis=1)


def _node(x03, agg23, batch, wroot, bconv, wih, bih, whh, bhh, w11, b11, w12,
          b12, w13, b13, w21, b21, w22, b22, w23, b23):
    n8 = x03.shape[0]
    d = 16
    n = n8 * 8
    g = 256
    batch2 = batch.reshape(1, n)
    return pl.pallas_call(
        _node_body,
        out_shape=jax.ShapeDtypeStruct((g, 2), jnp.float32),
    )(x03, agg23, batch2, wroot, bconv.reshape(1, d), wih,
      bih.reshape(1, 3 * d), whh, bhh.reshape(1, 3 * d), w11,
      b11.reshape(1, d), w12, b12.reshape(1, d), w13, b13.reshape(1, 1), w21,
      b21.reshape(1, d), w22, b22.reshape(1, d), w23, b23.reshape(1, 1))


def _v3(a):
    s = a.shape
    return a.reshape(s[:-2] + (s[-2] // 8, 8, s[-1]))


def kernel(x, edge_index, edge_attr, batch, W0, b0, We1, be1, We2, be2, Wroot,
           bconv, Wih, bih, Whh, bhh, W11, b11, W12, b12, W13, b13, W21, b21,
           W22, b22, W23, b23):
    n = x.shape[0]
    d = W0.shape[1]
    x03 = _lin0(x, W0, b0)
    src = edge_index[0]
    dst = edge_index[1]
    x0s = _sc_gather(x03.reshape(n, d), src)
    msg3 = _edge(_v3(edge_attr), _v3(x0s), We1, be1,
                 We2.reshape(d * d, d), be2.reshape(d, d))
    e = edge_attr.shape[0]
    agg2 = _sc_scatter_add(msg3.reshape(e, d), dst, n)
    return _node(x03, _v3(agg2), batch, Wroot, bconv, Wih, bih, Whh, bhh,
                 W11, b11, W12, b12, W13, b13, W21, b21, W22, b22, W23, b23)


# speedup vs baseline: 5.5288x; 1.2856x over previous
import functools

import jax
import jax.numpy as jnp
from jax import lax
from jax.experimental import pallas as pl
from jax.experimental.pallas import tpu as pltpu
from jax.experimental.pallas import tpu_sc as plsc


def _lin0_body(x_ref, w_ref, b_ref, o_ref):
    r = jnp.maximum(
        jnp.dot(x_ref[...], w_ref[...], preferred_element_type=jnp.float32)
        + b_ref[...], 0.0)
    bn, d = r.shape
    o_ref[...] = r.reshape(bn // 8, 8, d)


def _lin0(x, w, b):
    n, f = x.shape
    d = w.shape[1]
    return pl.pallas_call(
        _lin0_body,
        out_shape=jax.ShapeDtypeStruct((n // 8, 8, d), jnp.float32),
    )(x, w, b.reshape(1, d))


def _sc_gather(table, idx):
    e = idx.shape[0]
    d = table.shape[1]
    nw = 32
    bpw = e // nw
    mesh = plsc.VectorSubcoreMesh(core_axis_name="c", subcore_axis_name="s")

    @functools.partial(
        pl.kernel, mesh=mesh,
        out_type=jax.ShapeDtypeStruct((e, d), jnp.float32),
        compiler_params=pltpu.CompilerParams(use_tc_tiling_on_sc=False),
        scratch_types=[pltpu.VMEM((bpw,), jnp.int32),
                       pltpu.VMEM((bpw, d), jnp.float32),
                       pltpu.SemaphoreType.DMA],
    )
    def k(table_hbm, idx_hbm, out_hbm, idx_v, rows_v, sem):
        wid = lax.axis_index("s") * 2 + lax.axis_index("c")
        base = wid * bpw
        pltpu.sync_copy(idx_hbm.at[pl.ds(base, bpw)], idx_v)
        pltpu.async_copy(table_hbm.at[idx_v], rows_v, sem).wait()
        pltpu.sync_copy(rows_v, out_hbm.at[pl.ds(base, bpw)])

    return k(table, idx)


def _sc_scatter_add(msg, dst, n):
    e, d = msg.shape
    nw, ns = 32, 16
    bpw = e // nw
    nps = n // ns
    mesh = plsc.VectorSubcoreMesh(core_axis_name="c", subcore_axis_name="s")

    @functools.partial(
        pl.kernel, mesh=mesh,
        out_type=jax.ShapeDtypeStruct((2, n, d), jnp.float32),
        compiler_params=pltpu.CompilerParams(use_tc_tiling_on_sc=False),
        scratch_types=[pltpu.VMEM((bpw,), jnp.int32),
                       pltpu.VMEM((bpw, d), jnp.float32),
                       pltpu.VMEM((nps, d), jnp.float32),
                       pltpu.VMEM_SHARED((n, d), jnp.float32),
                       pltpu.SemaphoreType.DMA],
    )
    def k(msg_hbm, dst_hbm, zeros_hbm, out_hbm, idx_v, rows_v, z_v, acc_sh,
          sem):
        cid = lax.axis_index("c")
        sid = lax.axis_index("s")
        wid = sid * 2 + cid
        pltpu.sync_copy(zeros_hbm.at[pl.ds(sid * nps, nps)], z_v)
        pltpu.sync_copy(z_v, acc_sh.at[pl.ds(sid * nps, nps)])
        plsc.subcore_barrier()
        base = wid * bpw
        pltpu.sync_copy(dst_hbm.at[pl.ds(base, bpw)], idx_v)
        pltpu.sync_copy(msg_hbm.at[pl.ds(base, bpw)], rows_v)
        pltpu.sync_copy(rows_v, acc_sh.at[idx_v], add=True)
        plsc.subcore_barrier()
        pltpu.sync_copy(acc_sh.at[pl.ds(sid * nps, nps)],
                        out_hbm.at[cid, pl.ds(sid * nps, nps)])

    return k(msg, dst, jnp.zeros((n, d), jnp.float32))


def _edge_body(ea_ref, xs_ref, w1_ref, b1_ref, k8_ref, l8_ref, w2_ref,
               bm_ref, o_ref):
    ea8 = ea_ref[...]
    xs8 = xs_ref[...]
    eh8 = jnp.maximum(
        jnp.dot(ea8, w1_ref[...], preferred_element_type=jnp.float32)
        + b1_ref[...], 0.0)
    z8 = (jnp.dot(eh8, k8_ref[...], preferred_element_type=jnp.float32)
          * jnp.dot(xs8, l8_ref[...], preferred_element_type=jnp.float32))
    o_ref[...] = (jnp.dot(z8, w2_ref[...], preferred_element_type=jnp.float32)
                  + jnp.dot(xs8, bm_ref[...],
                            preferred_element_type=jnp.float32))


def _edge(ea8, xs8, bdw1, be1t, k8, l8, w2r8, bdbe2):
    e8 = ea8.shape[0]
    r8 = 400
    return pl.pallas_call(
        _edge_body,
        grid=(e8 // r8,),
        in_specs=[pl.BlockSpec((r8, 128), lambda i: (i, 0)),
                  pl.BlockSpec((r8, 128), lambda i: (i, 0)),
                  pl.BlockSpec((128, 128), lambda i: (0, 0)),
                  pl.BlockSpec((1, 128), lambda i: (0, 0)),
                  pl.BlockSpec((128, 2048), lambda i: (0, 0)),
                  pl.BlockSpec((128, 2048), lambda i: (0, 0)),
                  pl.BlockSpec((2048, 128), lambda i: (0, 0)),
                  pl.BlockSpec((128, 128), lambda i: (0, 0))],
        out_specs=pl.BlockSpec((r8, 128), lambda i: (i, 0)),
        out_shape=jax.ShapeDtypeStruct((e8, 128), jnp.float32),
    )(ea8, xs8, bdw1, be1t, k8, l8, w2r8, bdbe2)


def _node_body(x0_ref, agg_ref, b2_ref, wroot_ref, bconv_ref, wih_ref,
               bih_ref, whh_ref, bhh_ref, w11_ref, b11_ref, w12_ref, b12_ref,
               w13_ref, b13_ref, w21_ref, b21_ref, w22_ref, b22_ref, w23_ref,
               b23_ref, o_ref):
    r8 = x0_ref.shape[0]
    bn = r8 * 8
    x0 = x0_ref[...].reshape(bn, 16)
    agg = agg_ref[0].reshape(bn, 16) + agg_ref[1].reshape(bn, 16)
    xc = jnp.maximum(
        jnp.dot(x0, wroot_ref[...], preferred_element_type=jnp.float32)
        + agg + bconv_ref[...], 0.0)
    gi = jnp.dot(xc, wih_ref[...], preferred_element_type=jnp.float32) \
        + bih_ref[...]
    gh = jnp.dot(x0, whh_ref[...], preferred_element_type=jnp.float32) \
        + bhh_ref[...]
    r = jax.nn.sigmoid(gi[:, 0:16] + gh[:, 0:16])
    zg = jax.nn.sigmoid(gi[:, 16:32] + gh[:, 16:32])
    ng = jnp.tanh(gi[:, 32:48] + r * gh[:, 32:48])
    hn = (1.0 - zg) * ng + zg * x0
    bvals = b2_ref[...]
    g_iota = lax.broadcasted_iota(jnp.int32, (256, bn), 0)
    onehot = (g_iota == bvals).astype(jnp.float32)
    p = jnp.dot(onehot, hn, preferred_element_type=jnp.float32)
    x1 = jnp.maximum(
        jnp.dot(p, w11_ref[...], preferred_element_type=jnp.float32)
        + b11_ref[...], 0.0)
    x1 = jnp.maximum(
        jnp.dot(x1, w12_ref[...], preferred_element_type=jnp.float32)
        + b12_ref[...], 0.0)
    o1 = jnp.dot(x1, w13_ref[...], preferred_element_type=jnp.float32) \
        + b13_ref[...]
    x2 = jnp.maximum(
        jnp.dot(p, w21_ref[...], preferred_element_type=jnp.float32)
        + b21_ref[...], 0.0)
    x2 = jnp.maximum(
        jnp.dot(x2, w22_ref[...], preferred_element_type=jnp.float32)
        + b22_ref[...], 0.0)
    o2 = jnp.dot(x2, w23_ref[...], preferred_element_type=jnp.float32) \
        + b23_ref[...]
    o_ref[...] = jnp.concatenate([o1, o2], axis=1)


def _node(x03, agg23, batch, wroot, bconv, wih, bih, whh, bhh, w11, b11, w12,
          b12, w13, b13, w21, b21, w22, b22, w23, b23):
    n8 = x03.shape[0]
    d = 16
    n = n8 * 8
    g = 256
    batch2 = batch.reshape(1, n)
    return pl.pallas_call(
        _node_body,
        out_shape=jax.ShapeDtypeStruct((g, 2), jnp.float32),
    )(x03, agg23, batch2, wroot, bconv.reshape(1, d), wih,
      bih.reshape(1, 3 * d), whh, bhh.reshape(1, 3 * d), w11,
      b11.reshape(1, d), w12, b12.reshape(1, d), w13, b13.reshape(1, 1), w21,
      b21.reshape(1, d), w22, b22.reshape(1, d), w23, b23.reshape(1, 1))


def _v3(a):
    s = a.shape
    return a.reshape(s[:-2] + (s[-2] // 8, 8, s[-1]))


def kernel(x, edge_index, edge_attr, batch, W0, b0, We1, be1, We2, be2, Wroot,
           bconv, Wih, bih, Whh, bhh, W11, b11, W12, b12, W13, b13, W21, b21,
           W22, b22, W23, b23):
    n = x.shape[0]
    d = W0.shape[1]
    e = edge_attr.shape[0]
    x03 = _lin0(x, W0, b0)
    src = edge_index[0]
    dst = edge_index[1]
    x0s = _sc_gather(x03.reshape(n, d), src)

    eye8 = jnp.eye(8, dtype=jnp.float32)
    col = jnp.arange(d * d)[None, :]
    kmat = (jnp.arange(d)[:, None] == col // d).astype(jnp.float32)
    lmat = (jnp.arange(d)[:, None] == col % d).astype(jnp.float32)
    msg8 = _edge(edge_attr.reshape(e // 8, 128), x0s.reshape(e // 8, 128),
                 jnp.kron(eye8, We1), jnp.tile(be1, 8).reshape(1, 128),
                 jnp.kron(eye8, kmat), jnp.kron(eye8, lmat),
                 jnp.kron(eye8, We2.reshape(d * d, d)),
                 jnp.kron(eye8, be2.reshape(d, d)))

    agg2 = _sc_scatter_add(msg8.reshape(e, d), dst, n)
    return _node(x03, _v3(agg2), batch, Wroot, bconv, Wih, bih, Whh, bhh,
                 W11, b11, W12, b12, W13, b13, W21, b21, W22, b22, W23, b23)


# speedup vs baseline: 5.9655x; 1.0790x over previous
import functools

import jax
import jax.numpy as jnp
from jax import lax
from jax.experimental import pallas as pl
from jax.experimental.pallas import tpu as pltpu
from jax.experimental.pallas import tpu_sc as plsc


def _lin0_body(x_ref, w_ref, b_ref, o_ref):
    w = w_ref[...]
    b = b_ref[...]
    chunks = []
    for a in range(8):
        xa = x_ref[:, a, :]
        chunks.append(jnp.maximum(
            jnp.dot(xa, w, preferred_element_type=jnp.float32) + b, 0.0))
    o_ref[...] = jnp.concatenate(chunks, axis=1)


def _lin0(x, w, b):
    n, f = x.shape
    d = w.shape[1]
    return pl.pallas_call(
        _lin0_body,
        out_shape=jax.ShapeDtypeStruct((n // 8, 8 * d), jnp.float32),
    )(x.reshape(n // 8, 8, f), w, b.reshape(1, d))


def _sc_gather(table, edge_index):
    e = edge_index.shape[1]
    d = table.shape[1]
    nw = 32
    bpw = e // nw
    mesh = plsc.VectorSubcoreMesh(core_axis_name="c", subcore_axis_name="s")

    @functools.partial(
        pl.kernel, mesh=mesh,
        out_type=jax.ShapeDtypeStruct((e, d), jnp.float32),
        compiler_params=pltpu.CompilerParams(use_tc_tiling_on_sc=False),
        scratch_types=[pltpu.VMEM((bpw,), jnp.int32),
                       pltpu.VMEM((bpw, d), jnp.float32),
                       pltpu.SemaphoreType.DMA],
    )
    def k(table_hbm, ei_hbm, out_hbm, idx_v, rows_v, sem):
        wid = lax.axis_index("s") * 2 + lax.axis_index("c")
        base = wid * bpw
        pltpu.sync_copy(ei_hbm.at[0, pl.ds(base, bpw)], idx_v)
        pltpu.async_copy(table_hbm.at[idx_v], rows_v, sem).wait()
        pltpu.sync_copy(rows_v, out_hbm.at[pl.ds(base, bpw)])

    return k(table, edge_index)


def _sc_scatter_add(msg, edge_index, n):
    e, d = msg.shape
    nw, ns = 32, 16
    bpw = e // nw
    nps = n // ns
    mesh = plsc.VectorSubcoreMesh(core_axis_name="c", subcore_axis_name="s")

    @functools.partial(
        pl.kernel, mesh=mesh,
        out_type=jax.ShapeDtypeStruct((2, n, d), jnp.float32),
        compiler_params=pltpu.CompilerParams(use_tc_tiling_on_sc=False),
        scratch_types=[pltpu.VMEM((bpw,), jnp.int32),
                       pltpu.VMEM((bpw, d), jnp.float32),
                       pltpu.VMEM((nps, d), jnp.float32),
                       pltpu.VMEM_SHARED((n, d), jnp.float32),
                       pltpu.SemaphoreType.DMA],
    )
    def k(msg_hbm, ei_hbm, zeros_hbm, out_hbm, idx_v, rows_v, z_v, acc_sh,
          sem):
        cid = lax.axis_index("c")
        sid = lax.axis_index("s")
        wid = sid * 2 + cid
        pltpu.sync_copy(zeros_hbm.at[pl.ds(sid * nps, nps)], z_v)
        pltpu.sync_copy(z_v, acc_sh.at[pl.ds(sid * nps, nps)])
        plsc.subcore_barrier()
        base = wid * bpw
        pltpu.sync_copy(ei_hbm.at[1, pl.ds(base, bpw)], idx_v)
        pltpu.sync_copy(msg_hbm.at[pl.ds(base, bpw)], rows_v)
        pltpu.sync_copy(rows_v, acc_sh.at[idx_v], add=True)
        plsc.subcore_barrier()
        pltpu.sync_copy(acc_sh.at[pl.ds(sid * nps, nps)],
                        out_hbm.at[cid, pl.ds(sid * nps, nps)])

    return k(msg, edge_index, jnp.zeros((n, d), jnp.float32))


def _edge_body(ea_ref, xs_ref, w1_ref, b1_ref, k8_ref, l8_ref, w2_ref,
               bm_ref, o_ref):
    ea8 = ea_ref[...]
    xs8 = xs_ref[...]
    eh8 = jnp.maximum(
        jnp.dot(ea8, w1_ref[...], preferred_element_type=jnp.float32)
        + b1_ref[...], 0.0)
    z8 = (jnp.dot(eh8, k8_ref[...], preferred_element_type=jnp.float32)
          * jnp.dot(xs8, l8_ref[...], preferred_element_type=jnp.float32))
    o_ref[...] = (jnp.dot(z8, w2_ref[...], preferred_element_type=jnp.float32)
                  + jnp.dot(xs8, bm_ref[...],
                            preferred_element_type=jnp.float32))


def _edge(ea8, xs8, bdw1, be1t, k8, l8, w2r8, bdbe2):
    e8 = ea8.shape[0]
    r8 = 400
    return pl.pallas_call(
        _edge_body,
        grid=(e8 // r8,),
        in_specs=[pl.BlockSpec((r8, 128), lambda i: (i, 0)),
                  pl.BlockSpec((r8, 128), lambda i: (i, 0)),
                  pl.BlockSpec((128, 128), lambda i: (0, 0)),
                  pl.BlockSpec((1, 128), lambda i: (0, 0)),
                  pl.BlockSpec((128, 2048), lambda i: (0, 0)),
                  pl.BlockSpec((128, 2048), lambda i: (0, 0)),
                  pl.BlockSpec((2048, 128), lambda i: (0, 0)),
                  pl.BlockSpec((128, 128), lambda i: (0, 0))],
        out_specs=pl.BlockSpec((r8, 128), lambda i: (i, 0)),
        out_shape=jax.ShapeDtypeStruct((e8, 128), jnp.float32),
    )(ea8, xs8, bdw1, be1t, k8, l8, w2r8, bdbe2)


def _node_body(x0_ref, agg_ref, bt_ref, wroot_ref, bconv_ref, wir_ref,
               wiz_ref, win_ref, whr_ref, whz_ref, whn_ref, bir_ref, biz_ref,
               bin_ref, bhr_ref, bhz_ref, bhn_ref, w11_ref, b11_ref, w12_ref,
               b12_ref, w13_ref, b13_ref, w21_ref, b21_ref, w22_ref, b22_ref,
               w23_ref, b23_ref, o_ref):
    n8 = x0_ref.shape[0]
    x0 = x0_ref[...]
    agg = agg_ref[0] + agg_ref[1]
    xc = jnp.maximum(
        jnp.dot(x0, wroot_ref[...], preferred_element_type=jnp.float32)
        + agg + bconv_ref[...], 0.0)
    gir = jnp.dot(xc, wir_ref[...], preferred_element_type=jnp.float32) \
        + bir_ref[...]
    giz = jnp.dot(xc, wiz_ref[...], preferred_element_type=jnp.float32) \
        + biz_ref[...]
    gin = jnp.dot(xc, win_ref[...], preferred_element_type=jnp.float32) \
        + bin_ref[...]
    ghr = jnp.dot(x0, whr_ref[...], preferred_element_type=jnp.float32) \
        + bhr_ref[...]
    ghz = jnp.dot(x0, whz_ref[...], preferred_element_type=jnp.float32) \
        + bhz_ref[...]
    ghn = jnp.dot(x0, whn_ref[...], preferred_element_type=jnp.float32) \
        + bhn_ref[...]
    r = jax.nn.sigmoid(gir + ghr)
    zg = jax.nn.sigmoid(giz + ghz)
    ng = jnp.tanh(gin + r * ghn)
    hn = (1.0 - zg) * ng + zg * x0
    g_iota = lax.broadcasted_iota(jnp.int32, (256, n8), 0)
    p = jnp.zeros((256, 16), jnp.float32)
    for a in range(8):
        onehot = (g_iota == bt_ref[a:a + 1, :]).astype(jnp.float32)
        p = p + jnp.dot(onehot, hn[:, a * 16:(a + 1) * 16],
                        preferred_element_type=jnp.float32)
    x1 = jnp.maximum(
        jnp.dot(p, w11_ref[...], preferred_element_type=jnp.float32)
        + b11_ref[...], 0.0)
    x1 = jnp.maximum(
        jnp.dot(x1, w12_ref[...], preferred_element_type=jnp.float32)
        + b12_ref[...], 0.0)
    o1 = jnp.dot(x1, w13_ref[...], preferred_element_type=jnp.float32) \
        + b13_ref[...]
    x2 = jnp.maximum(
        jnp.dot(p, w21_ref[...], preferred_element_type=jnp.float32)
        + b21_ref[...], 0.0)
    x2 = jnp.maximum(
        jnp.dot(x2, w22_ref[...], preferred_element_type=jnp.float32)
        + b22_ref[...], 0.0)
    o2 = jnp.dot(x2, w23_ref[...], preferred_element_type=jnp.float32) \
        + b23_ref[...]
    o_ref[...] = jnp.concatenate([o1, o2], axis=1)


def _node(x08, agg28, batch, wroot, bconv, wih, bih, whh, bhh, w11, b11, w12,
          b12, w13, b13, w21, b21, w22, b22, w23, b23):
    n8 = x08.shape[0]
    d = 16
    g = 256
    eye8 = jnp.eye(8, dtype=jnp.float32)
    kr = lambda w: jnp.kron(eye8, w)
    t8 = lambda v: jnp.tile(v, 8).reshape(1, 128)
    bt = batch.reshape(n8, 8).T
    return pl.pallas_call(
        _node_body,
        out_shape=jax.ShapeDtypeStruct((g, 2), jnp.float32),
    )(x08, agg28, bt, kr(wroot), t8(bconv),
      kr(wih[:, 0:d]), kr(wih[:, d:2 * d]), kr(wih[:, 2 * d:3 * d]),
      kr(whh[:, 0:d]), kr(whh[:, d:2 * d]), kr(whh[:, 2 * d:3 * d]),
      t8(bih[0:d]), t8(bih[d:2 * d]), t8(bih[2 * d:3 * d]),
      t8(bhh[0:d]), t8(bhh[d:2 * d]), t8(bhh[2 * d:3 * d]),
      w11, b11.reshape(1, d), w12, b12.reshape(1, d), w13, b13.reshape(1, 1),
      w21, b21.reshape(1, d), w22, b22.reshape(1, d), w23, b23.reshape(1, 1))


def kernel(x, edge_index, edge_attr, batch, W0, b0, We1, be1, We2, be2, Wroot,
           bconv, Wih, bih, Whh, bhh, W11, b11, W12, b12, W13, b13, W21, b21,
           W22, b22, W23, b23):
    n = x.shape[0]
    d = W0.shape[1]
    e = edge_attr.shape[0]
    x08 = _lin0(x, W0, b0)
    x0s = _sc_gather(x08.reshape(n, d), edge_index)

    eye8 = jnp.eye(8, dtype=jnp.float32)
    col = jnp.arange(d * d)[None, :]
    kmat = (jnp.arange(d)[:, None] == col // d).astype(jnp.float32)
    lmat = (jnp.arange(d)[:, None] == col % d).astype(jnp.float32)
    msg8 = _edge(edge_attr.reshape(e // 8, 128), x0s.reshape(e // 8, 128),
                 jnp.kron(eye8, We1), jnp.tile(be1, 8).reshape(1, 128),
                 jnp.kron(eye8, kmat), jnp.kron(eye8, lmat),
                 jnp.kron(eye8, We2.reshape(d * d, d)),
                 jnp.kron(eye8, be2.reshape(d, d)))

    agg2 = _sc_scatter_add(msg8.reshape(e, d), edge_index, n)
    return _node(x08, agg2.reshape(2, n // 8, 128), batch, Wroot, bconv, Wih,
                 bih, Whh, bhh, W11, b11, W12, b12, W13, b13, W21, b21, W22,
                 b22, W23, b23)


# speedup vs baseline: 6.0894x; 1.0208x over previous
import functools

import jax
import jax.numpy as jnp
from jax import lax
from jax.experimental import pallas as pl
from jax.experimental.pallas import tpu as pltpu
from jax.experimental.pallas import tpu_sc as plsc


def _lin0_body(x_ref, w_ref, b_ref, o_ref):
    w = w_ref[...]
    b = b_ref[...]
    chunks = []
    for a in range(8):
        xa = x_ref[:, a, :]
        chunks.append(jnp.maximum(
            jnp.dot(xa, w, preferred_element_type=jnp.float32) + b, 0.0))
    o_ref[...] = jnp.concatenate(chunks, axis=1)


def _lin0(x, w, b):
    n, f = x.shape
    d = w.shape[1]
    return pl.pallas_call(
        _lin0_body,
        out_shape=jax.ShapeDtypeStruct((n // 8, 8 * d), jnp.float32),
    )(x.reshape(n // 8, 8, f), w, b.reshape(1, d))


def _sc_gather(table, edge_index):
    e = edge_index.shape[1]
    d = table.shape[1]
    nw = 32
    bpw = e // nw
    mesh = plsc.VectorSubcoreMesh(core_axis_name="c", subcore_axis_name="s")

    @functools.partial(
        pl.kernel, mesh=mesh,
        out_type=jax.ShapeDtypeStruct((e, d), jnp.float32),
        compiler_params=pltpu.CompilerParams(use_tc_tiling_on_sc=False),
        scratch_types=[pltpu.VMEM((bpw,), jnp.int32),
                       pltpu.VMEM((bpw, d), jnp.float32),
                       pltpu.SemaphoreType.DMA],
    )
    def k(table_hbm, ei_hbm, out_hbm, idx_v, rows_v, sem):
        wid = lax.axis_index("s") * 2 + lax.axis_index("c")
        base = wid * bpw
        pltpu.sync_copy(ei_hbm.at[0, pl.ds(base, bpw)], idx_v)
        pltpu.async_copy(table_hbm.at[idx_v], rows_v, sem).wait()
        pltpu.sync_copy(rows_v, out_hbm.at[pl.ds(base, bpw)])

    return k(table, edge_index)


def _sc_scatter_add(msg, edge_index, n):
    e, d = msg.shape
    nw, ns = 32, 16
    bpw = e // nw
    nps = n // ns
    mesh = plsc.VectorSubcoreMesh(core_axis_name="c", subcore_axis_name="s")

    @functools.partial(
        pl.kernel, mesh=mesh,
        out_type=jax.ShapeDtypeStruct((2, n, d), jnp.float32),
        compiler_params=pltpu.CompilerParams(use_tc_tiling_on_sc=False),
        scratch_types=[pltpu.VMEM((bpw,), jnp.int32),
                       pltpu.VMEM((bpw, d), jnp.float32),
                       pltpu.VMEM((nps, d), jnp.float32),
                       pltpu.VMEM_SHARED((n, d), jnp.float32),
                       pltpu.SemaphoreType.DMA],
    )
    def k(msg_hbm, ei_hbm, zeros_hbm, out_hbm, idx_v, rows_v, z_v, acc_sh,
          sem):
        cid = lax.axis_index("c")
        sid = lax.axis_index("s")
        wid = sid * 2 + cid
        pltpu.sync_copy(zeros_hbm.at[pl.ds(sid * nps, nps)], z_v)
        pltpu.sync_copy(z_v, acc_sh.at[pl.ds(sid * nps, nps)])
        plsc.subcore_barrier()
        base = wid * bpw
        pltpu.sync_copy(ei_hbm.at[1, pl.ds(base, bpw)], idx_v)
        pltpu.sync_copy(msg_hbm.at[pl.ds(base, bpw)], rows_v)
        pltpu.sync_copy(rows_v, acc_sh.at[idx_v], add=True)
        plsc.subcore_barrier()
        pltpu.sync_copy(acc_sh.at[pl.ds(sid * nps, nps)],
                        out_hbm.at[cid, pl.ds(sid * nps, nps)])

    return k(msg, edge_index, jnp.zeros((n, d), jnp.float32))


def _edge_body(ea_ref, xs_ref, w1_ref, b1_ref, k8_ref, l8_ref, w2_ref,
               bm_ref, o_ref):
    ea8 = ea_ref[...]
    xs8 = xs_ref[...]
    eh8 = jnp.maximum(
        jnp.dot(ea8, w1_ref[...], preferred_element_type=jnp.float32)
        + b1_ref[...], 0.0)
    z8 = (jnp.dot(eh8, k8_ref[...], preferred_element_type=jnp.float32)
          * jnp.dot(xs8, l8_ref[...], preferred_element_type=jnp.float32))
    o_ref[...] = (jnp.dot(z8, w2_ref[...], preferred_element_type=jnp.float32)
                  + jnp.dot(xs8, bm_ref[...],
                            preferred_element_type=jnp.float32))


def _edge(ea8, xs8, bdw1, be1t, k8, l8, w2r8, bdbe2):
    e8 = ea8.shape[0]
    r8 = 1000
    return pl.pallas_call(
        _edge_body,
        grid=(e8 // r8,),
        in_specs=[pl.BlockSpec((r8, 128), lambda i: (i, 0)),
                  pl.BlockSpec((r8, 128), lambda i: (i, 0)),
                  pl.BlockSpec((128, 128), lambda i: (0, 0)),
                  pl.BlockSpec((1, 128), lambda i: (0, 0)),
                  pl.BlockSpec((128, 2048), lambda i: (0, 0)),
                  pl.BlockSpec((128, 2048), lambda i: (0, 0)),
                  pl.BlockSpec((2048, 128), lambda i: (0, 0)),
                  pl.BlockSpec((128, 128), lambda i: (0, 0))],
        out_specs=pl.BlockSpec((r8, 128), lambda i: (i, 0)),
        out_shape=jax.ShapeDtypeStruct((e8, 128), jnp.float32),
    )(ea8, xs8, bdw1, be1t, k8, l8, w2r8, bdbe2)


def _node_body(x0_ref, agg_ref, bt_ref, wroot_ref, bconv_ref, wir_ref,
               wiz_ref, win_ref, whr_ref, whz_ref, whn_ref, bir_ref, biz_ref,
               bin_ref, bhr_ref, bhz_ref, bhn_ref, w11_ref, b11_ref, w12_ref,
               b12_ref, w13_ref, b13_ref, w21_ref, b21_ref, w22_ref, b22_ref,
               w23_ref, b23_ref, o_ref):
    n8 = x0_ref.shape[0]
    x0 = x0_ref[...]
    agg = agg_ref[0] + agg_ref[1]
    xc = jnp.maximum(
        jnp.dot(x0, wroot_ref[...], preferred_element_type=jnp.float32)
        + agg + bconv_ref[...], 0.0)
    gir = jnp.dot(xc, wir_ref[...], preferred_element_type=jnp.float32) \
        + bir_ref[...]
    giz = jnp.dot(xc, wiz_ref[...], preferred_element_type=jnp.float32) \
        + biz_ref[...]
    gin = jnp.dot(xc, win_ref[...], preferred_element_type=jnp.float32) \
        + bin_ref[...]
    ghr = jnp.dot(x0, whr_ref[...], preferred_element_type=jnp.float32) \
        + bhr_ref[...]
    ghz = jnp.dot(x0, whz_ref[...], preferred_element_type=jnp.float32) \
        + bhz_ref[...]
    ghn = jnp.dot(x0, whn_ref[...], preferred_element_type=jnp.float32) \
        + bhn_ref[...]
    r = jax.nn.sigmoid(gir + ghr)
    zg = jax.nn.sigmoid(giz + ghz)
    ng = jnp.tanh(gin + r * ghn)
    hn = (1.0 - zg) * ng + zg * x0
    g_iota = lax.broadcasted_iota(jnp.int32, (256, n8), 0)
    p = jnp.zeros((256, 16), jnp.float32)
    for a in range(8):
        onehot = (g_iota == bt_ref[a:a + 1, :]).astype(jnp.float32)
        p = p + jnp.dot(onehot, hn[:, a * 16:(a + 1) * 16],
                        preferred_element_type=jnp.float32)
    x1 = jnp.maximum(
        jnp.dot(p, w11_ref[...], preferred_element_type=jnp.float32)
        + b11_ref[...], 0.0)
    x1 = jnp.maximum(
        jnp.dot(x1, w12_ref[...], preferred_element_type=jnp.float32)
        + b12_ref[...], 0.0)
    o1 = jnp.dot(x1, w13_ref[...], preferred_element_type=jnp.float32) \
        + b13_ref[...]
    x2 = jnp.maximum(
        jnp.dot(p, w21_ref[...], preferred_element_type=jnp.float32)
        + b21_ref[...], 0.0)
    x2 = jnp.maximum(
        jnp.dot(x2, w22_ref[...], preferred_element_type=jnp.float32)
        + b22_ref[...], 0.0)
    o2 = jnp.dot(x2, w23_ref[...], preferred_element_type=jnp.float32) \
        + b23_ref[...]
    o_ref[...] = jnp.concatenate([o1, o2], axis=1)


def _node(x08, agg28, batch, wroot, bconv, wih, bih, whh, bhh, w11, b11, w12,
          b12, w13, b13, w21, b21, w22, b22, w23, b23):
    n8 = x08.shape[0]
    d = 16
    g = 256
    eye8 = jnp.eye(8, dtype=jnp.float32)
    kr = lambda w: jnp.kron(eye8, w)
    t8 = lambda v: jnp.tile(v, 8).reshape(1, 128)
    bt = batch.reshape(n8, 8).T
    return pl.pallas_call(
        _node_body,
        out_shape=jax.ShapeDtypeStruct((g, 2), jnp.float32),
    )(x08, agg28, bt, kr(wroot), t8(bconv),
      kr(wih[:, 0:d]), kr(wih[:, d:2 * d]), kr(wih[:, 2 * d:3 * d]),
      kr(whh[:, 0:d]), kr(whh[:, d:2 * d]), kr(whh[:, 2 * d:3 * d]),
      t8(bih[0:d]), t8(bih[d:2 * d]), t8(bih[2 * d:3 * d]),
      t8(bhh[0:d]), t8(bhh[d:2 * d]), t8(bhh[2 * d:3 * d]),
      w11, b11.reshape(1, d), w12, b12.reshape(1, d), w13, b13.reshape(1, 1),
      w21, b21.reshape(1, d), w22, b22.reshape(1, d), w23, b23.reshape(1, 1))


def kernel(x, edge_index, edge_attr, batch, W0, b0, We1, be1, We2, be2, Wroot,
           bconv, Wih, bih, Whh, bhh, W11, b11, W12, b12, W13, b13, W21, b21,
           W22, b22, W23, b23):
    n = x.shape[0]
    d = W0.shape[1]
    e = edge_attr.shape[0]
    x08 = _lin0(x, W0, b0)
    x0s = _sc_gather(x08.reshape(n, d), edge_index)

    eye8 = jnp.eye(8, dtype=jnp.float32)
    col = jnp.arange(d * d)[None, :]
    kmat = (jnp.arange(d)[:, None] == col // d).astype(jnp.float32)
    lmat = (jnp.arange(d)[:, None] == col % d).astype(jnp.float32)
    msg8 = _edge(edge_attr.reshape(e // 8, 128), x0s.reshape(e // 8, 128),
                 jnp.kron(eye8, We1), jnp.tile(be1, 8).reshape(1, 128),
                 jnp.kron(eye8, kmat), jnp.kron(eye8, lmat),
                 jnp.kron(eye8, We2.reshape(d * d, d)),
                 jnp.kron(eye8, be2.reshape(d, d)))

    agg2 = _sc_scatter_add(msg8.reshape(e, d), edge_index, n)
    return _node(x08, agg2.reshape(2, n // 8, 128), batch, Wroot, bconv, Wih,
                 bih, Whh, bhh, W11, b11, W12, b12, W13, b13, W21, b21, W22,
                 b22, W23, b23)


# speedup vs baseline: 6.0998x; 1.0017x over previous
import functools

import jax
import jax.numpy as jnp
from jax import lax
from jax.experimental import pallas as pl
from jax.experimental.pallas import tpu as pltpu
from jax.experimental.pallas import tpu_sc as plsc


def _lin0_body(x_ref, w_ref, b_ref, o_ref):
    w = w_ref[...]
    b = b_ref[...]
    chunks = []
    for a in range(8):
        xa = x_ref[:, a, :]
        chunks.append(jnp.maximum(
            jnp.dot(xa, w, preferred_element_type=jnp.float32) + b, 0.0))
    o_ref[...] = jnp.concatenate(chunks, axis=1)


def _lin0(x, w, b):
    n, f = x.shape
    d = w.shape[1]
    return pl.pallas_call(
        _lin0_body,
        out_shape=jax.ShapeDtypeStruct((n // 8, 8 * d), jnp.float32),
    )(x.reshape(n // 8, 8, f), w, b.reshape(1, d))


def _sc_gather(table, edge_index):
    e = edge_index.shape[1]
    n, d = table.shape
    nw = 32
    bpw = e // nw
    mesh = plsc.VectorSubcoreMesh(core_axis_name="c", subcore_axis_name="s")

    @functools.partial(
        pl.kernel, mesh=mesh,
        out_type=jax.ShapeDtypeStruct((e, d), jnp.float32),
        compiler_params=pltpu.CompilerParams(use_tc_tiling_on_sc=False),
        scratch_types=[pltpu.VMEM((bpw,), jnp.int32),
                       pltpu.VMEM((bpw, d), jnp.float32),
                       pltpu.SemaphoreType.DMA],
    )
    def k(table_hbm, ei_hbm, out_hbm, idx_v, rows_v, sem):
        wid = lax.axis_index("s") * 2 + lax.axis_index("c")
        base = wid * bpw
        pltpu.sync_copy(ei_hbm.at[0, pl.ds(base, bpw)], idx_v)
        pltpu.async_copy(table_hbm.at[idx_v], rows_v, sem).wait()
        pltpu.sync_copy(rows_v, out_hbm.at[pl.ds(base, bpw)])

    return k(table, edge_index)


def _sc_scatter_add(msg, edge_index, n):
    e, d = msg.shape
    nw, ns = 32, 16
    bpw = e // nw
    nps = n // ns
    mesh = plsc.VectorSubcoreMesh(core_axis_name="c", subcore_axis_name="s")

    @functools.partial(
        pl.kernel, mesh=mesh,
        out_type=jax.ShapeDtypeStruct((2, n, d), jnp.float32),
        compiler_params=pltpu.CompilerParams(use_tc_tiling_on_sc=False),
        scratch_types=[pltpu.VMEM((bpw,), jnp.int32),
                       pltpu.VMEM((bpw, d), jnp.float32),
                       pltpu.VMEM((nps, d), jnp.float32),
                       pltpu.VMEM_SHARED((n, d), jnp.float32),
                       pltpu.SemaphoreType.DMA],
    )
    def k(msg_hbm, ei_hbm, zeros_hbm, out_hbm, idx_v, rows_v, z_v, acc_sh,
          sem):
        cid = lax.axis_index("c")
        sid = lax.axis_index("s")
        wid = sid * 2 + cid
        pltpu.sync_copy(zeros_hbm.at[pl.ds(sid * nps, nps)], z_v)
        pltpu.sync_copy(z_v, acc_sh.at[pl.ds(sid * nps, nps)])
        plsc.subcore_barrier()
        base = wid * bpw
        pltpu.sync_copy(ei_hbm.at[1, pl.ds(base, bpw)], idx_v)
        pltpu.sync_copy(msg_hbm.at[pl.ds(base, bpw)], rows_v)
        pltpu.sync_copy(rows_v, acc_sh.at[idx_v], add=True)
        plsc.subcore_barrier()
        pltpu.sync_copy(acc_sh.at[pl.ds(sid * nps, nps)],
                        out_hbm.at[cid, pl.ds(sid * nps, nps)])

    return k(msg, edge_index, jnp.zeros((n, d), jnp.float32))


def _edge_body(ea_ref, xs_ref, w1_ref, b1_ref, k8_ref, l8_ref, w2_ref,
               bm_ref, o_ref):
    ea8 = ea_ref[...]
    xs8 = xs_ref[...]
    eh8 = jnp.maximum(
        jnp.dot(ea8, w1_ref[...], preferred_element_type=jnp.float32)
        + b1_ref[...], 0.0)
    z8 = (jnp.dot(eh8, k8_ref[...], preferred_element_type=jnp.float32)
          * jnp.dot(xs8, l8_ref[...], preferred_element_type=jnp.float32))
    o_ref[...] = (jnp.dot(z8, w2_ref[...], preferred_element_type=jnp.float32)
                  + jnp.dot(xs8, bm_ref[...],
                            preferred_element_type=jnp.float32))


def _edge(ea8, xs8, bdw1, be1t, k8, l8, w2r8, bdbe2):
    e8 = ea8.shape[0]
    r8 = 1000
    return pl.pallas_call(
        _edge_body,
        grid=(e8 // r8,),
        in_specs=[pl.BlockSpec((r8, 128), lambda i: (i, 0)),
                  pl.BlockSpec((r8, 128), lambda i: (i, 0)),
                  pl.BlockSpec((128, 128), lambda i: (0, 0)),
                  pl.BlockSpec((1, 128), lambda i: (0, 0)),
                  pl.BlockSpec((128, 2048), lambda i: (0, 0)),
                  pl.BlockSpec((128, 2048), lambda i: (0, 0)),
                  pl.BlockSpec((2048, 128), lambda i: (0, 0)),
                  pl.BlockSpec((128, 128), lambda i: (0, 0))],
        out_specs=pl.BlockSpec((r8, 128), lambda i: (i, 0)),
        out_shape=jax.ShapeDtypeStruct((e8, 128), jnp.float32),
    )(ea8, xs8, bdw1, be1t, k8, l8, w2r8, bdbe2)


def _node_body(x0_ref, agg_ref, bt_ref, wroot_ref, bconv_ref, wir_ref,
               wiz_ref, win_ref, whr_ref, whz_ref, whn_ref, bir_ref, biz_ref,
               bin_ref, bhr_ref, bhz_ref, bhn_ref, w11_ref, b11_ref, w12_ref,
               b12_ref, w13_ref, b13_ref, w21_ref, b21_ref, w22_ref, b22_ref,
               w23_ref, b23_ref, o_ref):
    n8 = x0_ref.shape[0]
    x0 = x0_ref[...]
    agg = agg_ref[0] + agg_ref[1]
    xc = jnp.maximum(
        jnp.dot(x0, wroot_ref[...], preferred_element_type=jnp.float32)
        + agg + bconv_ref[...], 0.0)
    gir = jnp.dot(xc, wir_ref[...], preferred_element_type=jnp.float32) \
        + bir_ref[...]
    giz = jnp.dot(xc, wiz_ref[...], preferred_element_type=jnp.float32) \
        + biz_ref[...]
    gin = jnp.dot(xc, win_ref[...], preferred_element_type=jnp.float32) \
        + bin_ref[...]
    ghr = jnp.dot(x0, whr_ref[...], preferred_element_type=jnp.float32) \
        + bhr_ref[...]
    ghz = jnp.dot(x0, whz_ref[...], preferred_element_type=jnp.float32) \
        + bhz_ref[...]
    ghn = jnp.dot(x0, whn_ref[...], preferred_element_type=jnp.float32) \
        + bhn_ref[...]
    r = jax.nn.sigmoid(gir + ghr)
    zg = jax.nn.sigmoid(giz + ghz)
    ng = jnp.tanh(gin + r * ghn)
    hn = (1.0 - zg) * ng + zg * x0
    g_iota = lax.broadcasted_iota(jnp.int32, (256, n8), 0)
    p = jnp.zeros((256, 16), jnp.float32)
    for a in range(8):
        onehot = (g_iota == bt_ref[a:a + 1, :]).astype(jnp.float32)
        p = p + jnp.dot(onehot, hn[:, a * 16:(a + 1) * 16],
                        preferred_element_type=jnp.float32)
    x1 = jnp.maximum(
        jnp.dot(p, w11_ref[...], preferred_element_type=jnp.float32)
        + b11_ref[...], 0.0)
    x1 = jnp.maximum(
        jnp.dot(x1, w12_ref[...], preferred_element_type=jnp.float32)
        + b12_ref[...], 0.0)
    o1 = jnp.dot(x1, w13_ref[...], preferred_element_type=jnp.float32) \
        + b13_ref[...]
    x2 = jnp.maximum(
        jnp.dot(p, w21_ref[...], preferred_element_type=jnp.float32)
        + b21_ref[...], 0.0)
    x2 = jnp.maximum(
        jnp.dot(x2, w22_ref[...], preferred_element_type=jnp.float32)
        + b22_ref[...], 0.0)
    o2 = jnp.dot(x2, w23_ref[...], preferred_element_type=jnp.float32) \
        + b23_ref[...]
    o_ref[...] = jnp.concatenate([o1, o2], axis=1)


def _node(x08, agg28, batch, wroot, bconv, wih, bih, whh, bhh, w11, b11, w12,
          b12, w13, b13, w21, b21, w22, b22, w23, b23):
    n8 = x08.shape[0]
    d = 16
    g = 256
    eye8 = jnp.eye(8, dtype=jnp.float32)
    kr = lambda w: jnp.kron(eye8, w)
    t8 = lambda v: jnp.tile(v, 8).reshape(1, 128)
    bt = batch.reshape(n8, 8).T
    return pl.pallas_call(
        _node_body,
        out_shape=jax.ShapeDtypeStruct((g, 2), jnp.float32),
    )(x08, agg28, bt, kr(wroot), t8(bconv),
      kr(wih[:, 0:d]), kr(wih[:, d:2 * d]), kr(wih[:, 2 * d:3 * d]),
      kr(whh[:, 0:d]), kr(whh[:, d:2 * d]), kr(whh[:, 2 * d:3 * d]),
      t8(bih[0:d]), t8(bih[d:2 * d]), t8(bih[2 * d:3 * d]),
      t8(bhh[0:d]), t8(bhh[d:2 * d]), t8(bhh[2 * d:3 * d]),
      w11, b11.reshape(1, d), w12, b12.reshape(1, d), w13, b13.reshape(1, 1),
      w21, b21.reshape(1, d), w22, b22.reshape(1, d), w23, b23.reshape(1, 1))


def kernel(x, edge_index, edge_attr, batch, W0, b0, We1, be1, We2, be2, Wroot,
           bconv, Wih, bih, Whh, bhh, W11, b11, W12, b12, W13, b13, W21, b21,
           W22, b22, W23, b23):
    n = x.shape[0]
    d = W0.shape[1]
    e = edge_attr.shape[0]
    x08 = _lin0(x, W0, b0)
    x0s = _sc_gather(x08.reshape(n, d), edge_index)

    eye8 = jnp.eye(8, dtype=jnp.float32)
    col = jnp.arange(d * d)[None, :]
    kmat = (jnp.arange(d)[:, None] == col // d).astype(jnp.float32)
    lmat = (jnp.arange(d)[:, None] == col % d).astype(jnp.float32)
    msg8 = _edge(edge_attr.reshape(e // 8, 128), x0s.reshape(e // 8, 128),
                 jnp.kron(eye8, We1), jnp.tile(be1, 8).reshape(1, 128),
                 jnp.kron(eye8, kmat), jnp.kron(eye8, lmat),
                 jnp.kron(eye8, We2.reshape(d * d, d)),
                 jnp.kron(eye8, be2.reshape(d, d)))

    agg2 = _sc_scatter_add(msg8.reshape(e, d), edge_index, n)
    return _node(x08, agg2.reshape(2, n // 8, 128), batch, Wroot, bconv, Wih,
                 bih, Whh, bhh, W11, b11, W12, b12, W13, b13, W21, b21, W22,
                 b22, W23, b23)
